# baseline (speedup 1.0000x reference)
import jax
import jax.numpy as jnp
from jax.experimental import pallas as pl
from jax.experimental.pallas import tpu as pltpu

_K = 1024
_D = 64
_HW = 1024


def _rowsum_sq_lanes(m):
    sq = m * m
    acc = sq[:, 0:8]
    for j in range(1, 8):
        acc = acc + sq[:, 8 * j:8 * j + 8]
    t = acc[:, 0:4] + acc[:, 4:8]
    t = t[:, 0:2] + t[:, 2:4]
    return t[:, 0:1] + t[:, 1:2]


def _colsum_sq_sublanes(xb):
    sq = xb * xb

    def chunk(c):
        s = lambda i: sq[8 * c + i:8 * c + i + 1, :]
        return (((s(0) + s(1)) + (s(2) + s(3)))
                + ((s(4) + s(5)) + (s(6) + s(7))))

    acc = chunk(0)
    for c in range(1, 8):
        acc = acc + chunk(c)
    return acc


def _succ(f):
    return jax.lax.bitcast_convert_type(
        jax.lax.bitcast_convert_type(f, jnp.int32) + 1, jnp.float32)


def _pred(f):
    return jax.lax.bitcast_convert_type(
        jax.lax.bitcast_convert_type(f, jnp.int32) - 1, jnp.float32)


def _issue(x_ref, emb_ref, dot_ref, x2_ref):
    xb = x_ref[0]
    x2_ref[...] = _colsum_sq_sublanes(xb)
    dot_ref[...] = jax.lax.dot_general(
        emb_ref[...], xb, (((1,), (0,)), ((), ())),
        preferred_element_type=jnp.float32)


def _epilogue(dot_ref, x2_ref, e2_ref, iota_ref, out_ref):
    b = (x2_ref[...] + e2_ref[...]) - 2.0 * dot_ref[...]
    bmin = jnp.min(b, axis=0, keepdims=True)
    cmin = jnp.maximum(bmin, 0.0)
    m = jnp.sqrt(cmin)
    B = m * _succ(m)
    for _ in range(3):
        up = _succ(B)
        B = jnp.where(jnp.sqrt(up) == m, up, B)
    for _ in range(3):
        B = jnp.where(jnp.sqrt(B) == m, B, _pred(B))
    B = jnp.maximum(B, cmin)
    cand = jnp.where(b <= B, iota_ref[...], float(_K))
    idx = jnp.min(cand, axis=0, keepdims=True)
    out_ref[0] = idx.astype(jnp.int32)


def _vq_body(xa_ref, xb_ref, emb_ref, oute_ref, outo_ref,
             dota_ref, dotb_ref, x2a_ref, x2b_ref, e2_ref, iota_ref):
    @pl.when(pl.program_id(0) == 0)
    def _init():
        e2_ref[...] = _rowsum_sq_lanes(emb_ref[...])
        iota_ref[...] = jax.lax.broadcasted_iota(
            jnp.int32, (_K, _HW), 0).astype(jnp.float32)

    _issue(xa_ref, emb_ref, dota_ref, x2a_ref)
    _epilogue(dotb_ref, x2b_ref, e2_ref, iota_ref, outo_ref)
    _issue(xb_ref, emb_ref, dotb_ref, x2b_ref)
    _epilogue(dota_ref, x2a_ref, e2_ref, iota_ref, oute_ref)


def kernel(x, embeddings):
    B, d, H, W = x.shape
    hw = H * W
    x3 = x.reshape(B, d, hw)
    nb2 = B // 2
    oute, outo = pl.pallas_call(
        _vq_body,
        grid=(nb2 + 1,),
        in_specs=[
            pl.BlockSpec((1, d, hw), lambda j: (jnp.minimum(2 * j, B - 1), 0, 0)),
            pl.BlockSpec((1, d, hw), lambda j: (jnp.minimum(2 * j + 1, B - 1), 0, 0)),
            pl.BlockSpec((_K, d), lambda j: (0, 0)),
        ],
        out_specs=[
            pl.BlockSpec((1, 1, hw), lambda j: (j, 0, 0)),
            pl.BlockSpec((1, 1, hw), lambda j: (jnp.maximum(j - 1, 0), 0, 0)),
        ],
        out_shape=[
            jax.ShapeDtypeStruct((nb2 + 1, 1, hw), jnp.int32),
            jax.ShapeDtypeStruct((nb2, 1, hw), jnp.int32),
        ],
        scratch_shapes=[
            pltpu.VMEM((_K, _HW), jnp.float32),
            pltpu.VMEM((_K, _HW), jnp.float32),
            pltpu.VMEM((1, _HW), jnp.float32),
            pltpu.VMEM((1, _HW), jnp.float32),
            pltpu.VMEM((_K, 1), jnp.float32),
            pltpu.VMEM((_K, _HW), jnp.float32),
        ],
    )(x3, x3, embeddings)
    out = jnp.stack([oute[:nb2, 0], outo[:, 0]], axis=1)
    return out.reshape(B, H, W)
